# Initial kernel scaffold; baseline (speedup 1.0000x reference)
import jax
import jax.numpy as jnp
from jax.experimental import pallas as pl


def kernel(x, edge_index, W1, att_src1, att_dst1, b1, W2, att_src2, att_dst2, b2):
    raise NotImplementedError("write your pallas kernel here")


# speedup vs baseline: 1.1244x; 1.1244x over previous
import functools

import jax
import jax.numpy as jnp
from jax.experimental import pallas as pl
from jax.experimental.pallas import tpu as pltpu

N = 10000
E = 320000
D_IN = 128
HID = 64
HEADS = 8
D_OUT = 128

NPAD = 10240
BLK = 1024


def _mm_body(x_ref, w_ref, o_ref):
    o_ref[...] = jnp.dot(x_ref[...], w_ref[...],
                         preferred_element_type=jnp.float32)


def _pallas_mm(x, w):
    m, k = x.shape
    k2, n = w.shape
    xp = jnp.zeros((NPAD, k), x.dtype).at[:m].set(x)
    out = pl.pallas_call(
        _mm_body,
        grid=(NPAD // BLK,),
        in_specs=[
            pl.BlockSpec((BLK, k), lambda i: (i, 0)),
            pl.BlockSpec((k2, n), lambda i: (0, 0)),
        ],
        out_specs=pl.BlockSpec((BLK, n), lambda i: (i, 0)),
        out_shape=jax.ShapeDtypeStruct((NPAD, n), jnp.float32),
    )(xp, w)
    return out[:m]


def kernel(x, edge_index, W1, att_src1, att_dst1, b1, W2, att_src2, att_dst2, b2):
    loops = jnp.arange(N)
    ei = jnp.concatenate([edge_index, jnp.stack([loops, loops])], axis=1)
    src, dst = ei[0], ei[1]

    h = _pallas_mm(x, W1).reshape(N, HEADS, HID)
    a_s = jnp.sum(h * att_src1, axis=-1)
    a_d = jnp.sum(h * att_dst1, axis=-1)
    alpha = jax.nn.leaky_relu(a_s[src] + a_d[dst], 0.2)
    w = jnp.exp(alpha)
    denom = jax.ops.segment_sum(w, dst, num_segments=N)
    msg = h[src] * w[:, :, None]
    out = jax.ops.segment_sum(msg, dst, num_segments=N)
    out = out / (denom[:, :, None] + 1e-16)
    h1 = out.reshape(N, HEADS * HID) + b1
    h1 = jax.nn.elu(h1)

    h2 = _pallas_mm(h1, W2)
    a_s2 = jnp.sum(h2 * att_src2[0], axis=-1)
    a_d2 = jnp.sum(h2 * att_dst2[0], axis=-1)
    alpha2 = jax.nn.leaky_relu(a_s2[src] + a_d2[dst], 0.2)
    w2 = jnp.exp(alpha2)
    denom2 = jax.ops.segment_sum(w2, dst, num_segments=N)
    msg2 = h2[src] * w2[:, None]
    out2 = jax.ops.segment_sum(msg2, dst, num_segments=N)
    out2 = out2 / (denom2[:, None] + 1e-16)
    return out2 + b2


# speedup vs baseline: 3.0960x; 2.7536x over previous
import functools

import jax
import jax.numpy as jnp
from jax import lax
from jax.experimental import pallas as pl
from jax.experimental.pallas import tpu as pltpu
from jax.experimental.pallas import tpu_sc as plsc

N = 10000
E = 320000
D_IN = 128
HID = 64
HEADS = 8
D_OUT = 128

E2 = E + N
E2P = 360448
PADN = E2P - E2
SUP = 4
BLKR = 1000
NP = 10240
AW = 144

f32 = jnp.float32
i32 = jnp.int32

_GDN = lax.GatherDimensionNumbers(
    offset_dims=(), collapsed_slice_dims=(0,), start_index_map=(0,))


def _take16(v, const_idx):
    return lax.gather(v, const_idx[:, None], _GDN, slice_sizes=(1,),
                      mode=lax.GatherScatterMode.PROMISE_IN_BOUNDS)


def _make_sc1(np_, e2p):
    rpt = np_ // 16
    tpe = e2p // 16
    nsup = tpe // (SUP * 128)
    mesh = plsc.VectorSubcoreMesh(core_axis_name="c", subcore_axis_name="s")

    @functools.partial(
        pl.kernel,
        out_type=jax.ShapeDtypeStruct((4, N, AW), f32),
        mesh=mesh,
        compiler_params=pltpu.CompilerParams(use_tc_tiling_on_sc=False),
        scratch_types=[
            pltpu.VMEM_SHARED((N, AW), f32),
            pltpu.VMEM((SUP, 128), i32),
            pltpu.VMEM((SUP, 128), i32),
            pltpu.VMEM((SUP, 1024), f32),
            pltpu.VMEM((128, 128), f32),
            pltpu.VMEM((128, AW), f32),
            pltpu.SemaphoreType.DMA,
        ],
    )
    def sc1(srcr2, dstr2, wpre2d, h0, h1, h2r, h3, zac,
            out1,
            acc, sidx, didx, wb, hb, mb, sem_h):
        core = lax.axis_index("c")
        sub = lax.axis_index("s")
        iota = jnp.arange(16, dtype=i32)
        r0 = sub * rpt

        hrefs = (h0, h1, h2r, h3)
        for g in range(4):
            @pl.when(core == g // 2)
            def _pass(g=g):
                h_ref = hrefs[g]
                for kk in range(rpt // 128):
                    @pl.when(jnp.logical_or(sub < 15, kk < 3))
                    def _z(kk=kk):
                        pltpu.sync_copy(zac.at[pl.ds(r0 + kk * 128, 128)],
                                        acc.at[pl.ds(r0 + kk * 128, 128)])
                @pl.when(sub == 15)
                def _zt():
                    pltpu.sync_copy(zac.at[pl.ds(9984, 16)],
                                    acc.at[pl.ds(9984, 16)])
                plsc.subcore_barrier()

                idxc = []
                for parity in range(2):
                    idxc.append([
                        jnp.full((16,), parity * 8 + 2 * g, dtype=i32),
                        jnp.full((16,), parity * 8 + 2 * g + 1, dtype=i32),
                    ])

                def _sup(sc_, _):
                    ebase = sub * tpe + sc_ * (SUP * 128)
                    rbase = ebase // 128
                    pltpu.sync_copy(srcr2.at[pl.ds(rbase, SUP)], sidx)
                    pltpu.sync_copy(dstr2.at[pl.ds(rbase, SUP)], didx)
                    pltpu.sync_copy(wpre2d.at[pl.ds(rbase, SUP)], wb)
                    for j in range(SUP):
                        pltpu.async_copy(h_ref.at[sidx.at[j]], hb,
                                         sem_h).wait()

                        def _pair(q, _, j=j):
                            wv = wb[j, pl.ds(16 * q, 16)]
                            for parity in range(2):
                                e = 2 * q + parity
                                w0 = _take16(wv, idxc[parity][0])
                                w1 = _take16(wv, idxc[parity][1])
                                for jj in range(8):
                                    hv = hb[e, pl.ds(16 * jj, 16)]
                                    mb[e, pl.ds(16 * jj, 16)] = (
                                        hv * (w0 if jj < 4 else w1))
                                wz = jnp.where(iota == 0, w0,
                                               jnp.where(iota == 1, w1, 0.0))
                                mb[e, pl.ds(128, 16)] = wz
                            return 0
                        lax.fori_loop(0, 64, _pair, 0)
                        pltpu.sync_copy(mb, acc.at[didx.at[j]], add=True)
                    return 0
                lax.fori_loop(0, nsup, _sup, 0)
                plsc.subcore_barrier()
                for kk in range(rpt // 128):
                    @pl.when(jnp.logical_or(sub < 15, kk < 3))
                    def _d(kk=kk):
                        pltpu.sync_copy(
                            acc.at[pl.ds(r0 + kk * 128, 128)],
                            out1.at[g, pl.ds(r0 + kk * 128, 128)])
                @pl.when(sub == 15)
                def _dt(g=g):
                    pltpu.sync_copy(acc.at[pl.ds(9984, 16)],
                                    out1.at[g, pl.ds(9984, 16)])
    return sc1


def _make_sc2(np_, e2p):
    rpt = np_ // 16
    tpe = e2p // 32
    nsup = tpe // 1024
    mesh = plsc.VectorSubcoreMesh(core_axis_name="c", subcore_axis_name="s")

    @functools.partial(
        pl.kernel,
        out_type=jax.ShapeDtypeStruct((2, N, AW), f32),
        mesh=mesh,
        compiler_params=pltpu.CompilerParams(use_tc_tiling_on_sc=False),
        scratch_types=[
            pltpu.VMEM_SHARED((N, AW), f32),
            pltpu.VMEM((8, 128), i32),
            pltpu.VMEM((8, 128), i32),
            pltpu.VMEM((1, 1024), f32),
            pltpu.VMEM((128, 128), f32),
            pltpu.VMEM((128, AW), f32),
            pltpu.SemaphoreType.DMA,
        ],
    )
    def sc2(srcr2, dstr2, w2d, h2in, zac,
            accout,
            acc2, sidx, didx, wb, hb, mb, sem_h):
        core = lax.axis_index("c")
        sub = lax.axis_index("s")
        tile = core * 16 + sub
        iota = jnp.arange(16, dtype=i32)
        r0 = sub * rpt
        for kk in range(rpt // 128):
            @pl.when(jnp.logical_or(sub < 15, kk < 3))
            def _z(kk=kk):
                pltpu.sync_copy(zac.at[pl.ds(r0 + kk * 128, 128)],
                                acc2.at[pl.ds(r0 + kk * 128, 128)])
        @pl.when(sub == 15)
        def _zt():
            pltpu.sync_copy(zac.at[pl.ds(9984, 16)],
                            acc2.at[pl.ds(9984, 16)])
        plsc.subcore_barrier()

        lconst = [jnp.full((16,), l, dtype=i32) for l in range(16)]

        def _sup(sc_, _):
            ebase = tile * tpe + sc_ * 1024
            rbase = ebase // 128
            pltpu.sync_copy(srcr2.at[pl.ds(rbase, 8)], sidx)
            pltpu.sync_copy(dstr2.at[pl.ds(rbase, 8)], didx)
            pltpu.sync_copy(w2d.at[pl.ds(ebase // 1024, 1)], wb)
            for j in range(8):
                pltpu.async_copy(h2in.at[sidx.at[j]], hb, sem_h).wait()

                def _q(q, _, j=j):
                    wv = wb[0, pl.ds(j * 128 + 16 * q, 16)]
                    for l in range(16):
                        e = 16 * q + l
                        ws = _take16(wv, lconst[l])
                        for jj in range(8):
                            hv = hb[e, pl.ds(16 * jj, 16)]
                            mb[e, pl.ds(16 * jj, 16)] = hv * ws
                        wz = jnp.where(iota == 0, ws, 0.0)
                        mb[e, pl.ds(128, 16)] = wz
                    return 0
                lax.fori_loop(0, 8, _q, 0)
                pltpu.sync_copy(mb, acc2.at[didx.at[j]], add=True)
            return 0
        lax.fori_loop(0, nsup, _sup, 0)
        plsc.subcore_barrier()
        for kk in range(rpt // 128):
            @pl.when(jnp.logical_or(sub < 15, kk < 3))
            def _d(kk=kk):
                pltpu.sync_copy(
                    acc2.at[pl.ds(r0 + kk * 128, 128)],
                    accout.at[core, pl.ds(r0 + kk * 128, 128)])
        @pl.when(sub == 15)
        def _dt():
            pltpu.sync_copy(acc2.at[pl.ds(9984, 16)],
                            accout.at[core, pl.ds(9984, 16)])
    return sc2


def _tc1_body(x_ref, w1_ref, as_ref, ad_ref,
              h0_ref, h1_ref, h2_ref, h3_ref, as_o, ad_o):
    h = jnp.dot(x_ref[...], w1_ref[...], preferred_element_type=f32)
    h0_ref[...] = h[:, 0:128]
    h1_ref[...] = h[:, 128:256]
    h2_ref[...] = h[:, 256:384]
    h3_ref[...] = h[:, 384:512]
    as_o[...] = jnp.dot(h, as_ref[...], preferred_element_type=f32)
    ad_o[...] = jnp.dot(h, ad_ref[...], preferred_element_type=f32)


def _tcw_body(a_ref, w_ref):
    a = a_ref[...]
    a = jnp.where(a >= 0, a, 0.2 * a)
    w_ref[...] = jnp.exp(a)


def _tc2_body(o_ref, exp8_ref, b1_ref, w2_ref, att_ref, h2_o, a2_o):
    o = o_ref[...]
    hcat = jnp.concatenate([o[0][:, 0:128], o[1][:, 0:128],
                            o[2][:, 0:128], o[3][:, 0:128]], axis=-1)
    den8 = jnp.concatenate([o[0][:, 128:130], o[1][:, 128:130],
                            o[2][:, 128:130], o[3][:, 128:130]], axis=-1)
    den_rep = jnp.dot(den8, exp8_ref[...], preferred_element_type=f32)
    h1 = hcat / (den_rep + 1e-16) + b1_ref[...]
    h1 = jnp.where(h1 > 0, h1, jnp.exp(h1) - 1.0)
    h2 = jnp.dot(h1, w2_ref[...], preferred_element_type=f32)
    h2_o[...] = h2
    a2_o[...] = jnp.dot(h2, att_ref[...], preferred_element_type=f32)


def _tc3_body(acc_ref, b2_ref, out_ref):
    s = acc_ref[0][:, 0:128] + acc_ref[1][:, 0:128]
    d = acc_ref[0][:, 128:129] + acc_ref[1][:, 128:129] + 1e-16
    out_ref[...] = s / d + b2_ref[...]


def kernel(x, edge_index, W1, att_src1, att_dst1, b1, W2, att_src2, att_dst2, b2):
    ei = edge_index.astype(i32)
    loops = jnp.arange(N, dtype=i32)
    src_p = jnp.concatenate([ei[0], loops, jnp.zeros((PADN,), i32)])
    dst_p = jnp.concatenate([ei[1], loops, jnp.zeros((PADN,), i32)])
    srcr2 = src_p.reshape(E2P // 128, 128)
    dstr2 = dst_p.reshape(E2P // 128, 128)
    emask = (jnp.arange(E2P) < E2)

    eye8 = jnp.eye(8, dtype=f32)
    A_src = (att_src1[:, :, None] * eye8[:, None, :]).reshape(HEADS * HID, 8)
    A_dst = (att_dst1[:, :, None] * eye8[:, None, :]).reshape(HEADS * HID, 8)
    EXP8 = jnp.repeat(eye8, HID, axis=1)
    att2 = jnp.stack([att_src2[0], att_dst2[0]], axis=1)
    b1r = b1.reshape(1, HEADS * HID)
    b2r = b2.reshape(1, D_OUT)
    zac = jnp.zeros((N, AW), f32)

    grid = (N // BLKR,)
    h0, h1, h2g, h3, a_s, a_d = pl.pallas_call(
        _tc1_body,
        grid=grid,
        in_specs=[
            pl.BlockSpec((BLKR, D_IN), lambda i: (i, 0)),
            pl.BlockSpec((D_IN, HEADS * HID), lambda i: (0, 0)),
            pl.BlockSpec((HEADS * HID, 8), lambda i: (0, 0)),
            pl.BlockSpec((HEADS * HID, 8), lambda i: (0, 0)),
        ],
        out_specs=[pl.BlockSpec((BLKR, 128), lambda i: (i, 0))] * 4
        + [pl.BlockSpec((BLKR, 8), lambda i: (i, 0))] * 2,
        out_shape=[jax.ShapeDtypeStruct((N, 128), f32)] * 4
        + [jax.ShapeDtypeStruct((N, 8), f32)] * 2,
    )(x, W1, A_src, A_dst)

    alpha1 = jnp.take(a_s, src_p, axis=0) + jnp.take(a_d, dst_p, axis=0)
    alpha1 = jnp.where(emask[:, None], alpha1, -1e30)
    WROWS = E2P * 8 // 1024
    wpre2d = pl.pallas_call(
        _tcw_body,
        grid=(8,),
        in_specs=[pl.BlockSpec((WROWS // 8, 1024), lambda i: (i, 0))],
        out_specs=pl.BlockSpec((WROWS // 8, 1024), lambda i: (i, 0)),
        out_shape=jax.ShapeDtypeStruct((WROWS, 1024), f32),
    )(alpha1.reshape(WROWS, 1024))

    sc1 = _make_sc1(NP, E2P)
    out1 = sc1(srcr2, dstr2, wpre2d, h0, h1, h2g, h3, zac)

    h2, a2 = pl.pallas_call(
        _tc2_body,
        grid=grid,
        in_specs=[
            pl.BlockSpec((4, BLKR, AW), lambda i: (0, i, 0)),
            pl.BlockSpec((8, HEADS * HID), lambda i: (0, 0)),
            pl.BlockSpec((1, HEADS * HID), lambda i: (0, 0)),
            pl.BlockSpec((HEADS * HID, D_OUT), lambda i: (0, 0)),
            pl.BlockSpec((D_OUT, 2), lambda i: (0, 0)),
        ],
        out_specs=[pl.BlockSpec((BLKR, D_OUT), lambda i: (i, 0)),
                   pl.BlockSpec((BLKR, 2), lambda i: (i, 0))],
        out_shape=[jax.ShapeDtypeStruct((N, D_OUT), f32),
                   jax.ShapeDtypeStruct((N, 2), f32)],
    )(out1, EXP8, b1r, W2, att2)

    alpha2 = jnp.take(a2[:, 0], src_p) + jnp.take(a2[:, 1], dst_p)
    alpha2 = jnp.where(emask, alpha2, -1e30)
    W2ROWS = E2P // 1024
    w2d = pl.pallas_call(
        _tcw_body,
        grid=(1,),
        in_specs=[pl.BlockSpec((W2ROWS, 1024), lambda i: (0, 0))],
        out_specs=pl.BlockSpec((W2ROWS, 1024), lambda i: (0, 0)),
        out_shape=jax.ShapeDtypeStruct((W2ROWS, 1024), f32),
    )(alpha2.reshape(W2ROWS, 1024))

    sc2 = _make_sc2(NP, E2P)
    acc2 = sc2(srcr2, dstr2, w2d, h2, zac)

    out = pl.pallas_call(
        _tc3_body,
        grid=grid,
        in_specs=[
            pl.BlockSpec((2, BLKR, AW), lambda i: (0, i, 0)),
            pl.BlockSpec((1, D_OUT), lambda i: (0, 0)),
        ],
        out_specs=pl.BlockSpec((BLKR, D_OUT), lambda i: (i, 0)),
        out_shape=jax.ShapeDtypeStruct((N, D_OUT), f32),
    )(acc2, b2r)
    return out
